# Initial kernel scaffold; baseline (speedup 1.0000x reference)
import jax
import jax.numpy as jnp
from jax.experimental import pallas as pl


def kernel(data_base, edge_index_base, batch_base, data_residual, edge_index_residual, batch_residual, params):
    raise NotImplementedError("write your pallas kernel here")


# speedup vs baseline: 4.0426x; 4.0426x over previous
import functools

import jax
import jax.numpy as jnp
from jax import lax
from jax.experimental import pallas as pl
from jax.experimental.pallas import tpu as pltpu
from jax.experimental.pallas import tpu_sc as plsc

N = 50000
E = 800000
F_IN = 6
D1 = 32
D2 = 16
G = 64

NC = 2
NS = 16
NW = NC * NS
CH = 128
NCHUNK = -(-(E // NW) // CH)
EPW = NCHUNK * CH
EPAD = EPW * NW
NACC = 50048
ZR = NACC // NS
R = 2000
NBLK = N // R


def _make_seg_sum(D):
    mesh = plsc.VectorSubcoreMesh(core_axis_name="c", subcore_axis_name="s",
                                  num_cores=NC, num_subcores=NS)

    @functools.partial(
        pl.kernel,
        out_type=jax.ShapeDtypeStruct((NC, NACC, D), jnp.float32),
        mesh=mesh,
        scratch_types=[
            pltpu.VMEM((CH,), jnp.int32),
            pltpu.VMEM((CH,), jnp.int32),
            pltpu.VMEM((CH, D), jnp.float32),
            pltpu.VMEM_SHARED((NACC, D), jnp.float32),
        ],
        compiler_params=pltpu.CompilerParams(use_tc_tiling_on_sc=False),
    )
    def seg_sum(h_hbm, src_hbm, dst_hbm, zeros_hbm, out_hbm, sidx, didx, rows, acc):
        c = lax.axis_index("c")
        s = lax.axis_index("s")
        wid = s * NC + c
        pltpu.sync_copy(zeros_hbm, acc.at[pl.ds(s * ZR, ZR)])
        plsc.subcore_barrier()
        base = wid * EPW

        @pl.loop(0, NCHUNK)
        def _(i):
            off = base + i * CH
            pltpu.sync_copy(src_hbm.at[pl.ds(off, CH)], sidx)
            pltpu.sync_copy(dst_hbm.at[pl.ds(off, CH)], didx)
            pltpu.sync_copy(h_hbm.at[sidx], rows)
            pltpu.sync_copy(rows, acc.at[didx], add=True)

        plsc.subcore_barrier()
        pltpu.sync_copy(acc.at[pl.ds(s * ZR, ZR)], out_hbm.at[c, pl.ds(s * ZR, ZR)])

    return seg_sum


_make_seg_sum = functools.lru_cache(maxsize=None)(_make_seg_sum)


def _seg_sum16(h, src, dst, z):
    return _make_seg_sum(16)(h, src, dst, z)


def _seg_sum32(h, src, dst, z):
    return _make_seg_sum(32)(h, src, dst, z)


def _mlp_body(h_ref, agg_ref, w1_ref, b1_ref, w2_ref, b2_ref, t2_ref, stat_ref):
    x = h_ref[...] + agg_ref[0] + agg_ref[1]
    t = jnp.maximum(
        jnp.dot(x, w1_ref[...], preferred_element_type=jnp.float32) + b1_ref[...], 0.0)
    t2 = jnp.maximum(
        jnp.dot(t, w2_ref[...], preferred_element_type=jnp.float32) + b2_ref[...], 0.0)
    t2_ref[...] = t2

    @pl.when(pl.program_id(0) == 0)
    def _():
        stat_ref[...] = jnp.zeros_like(stat_ref)

    stat_ref[0:1, :] += jnp.sum(t2, axis=0, keepdims=True)


def _mlp(h, agg, w1, b1, w2, b2):
    din = h.shape[1]
    return pl.pallas_call(
        _mlp_body,
        grid=(NBLK,),
        in_specs=[
            pl.BlockSpec((R, din), lambda i: (i, 0)),
            pl.BlockSpec((NC, R, din), lambda i: (0, i, 0)),
            pl.BlockSpec((din, D1), lambda i: (0, 0)),
            pl.BlockSpec((1, D1), lambda i: (0, 0)),
            pl.BlockSpec((D1, D1), lambda i: (0, 0)),
            pl.BlockSpec((1, D1), lambda i: (0, 0)),
        ],
        out_specs=[
            pl.BlockSpec((R, D1), lambda i: (i, 0)),
            pl.BlockSpec((8, D1), lambda i: (0, 0)),
        ],
        out_shape=[
            jax.ShapeDtypeStruct((N, D1), jnp.float32),
            jax.ShapeDtypeStruct((8, D1), jnp.float32),
        ],
        compiler_params=pltpu.CompilerParams(dimension_semantics=("arbitrary",)),
    )(h, agg, w1, b1, w2, b2)


def _bnstat_body(t2_ref, stat_ref, out_ref):
    mean = stat_ref[0:1, :] / N

    @pl.when(pl.program_id(0) == 0)
    def _():
        out_ref[...] = jnp.zeros_like(out_ref)

    d = t2_ref[...] - mean
    out_ref[0:1, :] += jnp.sum(d * d, axis=0, keepdims=True)


def _bnstat(t2, stat):
    return pl.pallas_call(
        _bnstat_body,
        grid=(NBLK,),
        in_specs=[
            pl.BlockSpec((R, D1), lambda i: (i, 0)),
            pl.BlockSpec((8, D1), lambda i: (0, 0)),
        ],
        out_specs=pl.BlockSpec((8, D1), lambda i: (0, 0)),
        out_shape=jax.ShapeDtypeStruct((8, D1), jnp.float32),
        compiler_params=pltpu.CompilerParams(dimension_semantics=("arbitrary",)),
    )(t2, stat)


def _bn_body(t2_ref, stat_ref, cstat_ref, g_ref, b_ref, out_ref):
    mean = stat_ref[0:1, :] / N
    var = cstat_ref[0:1, :] / N
    inv = lax.rsqrt(var + 1e-5)
    out_ref[...] = (t2_ref[...] - mean) * inv * g_ref[...] + b_ref[...]


def _bn(t2, stat, cstat, g, b):
    return pl.pallas_call(
        _bn_body,
        grid=(NBLK,),
        in_specs=[
            pl.BlockSpec((R, D1), lambda i: (i, 0)),
            pl.BlockSpec((8, D1), lambda i: (0, 0)),
            pl.BlockSpec((8, D1), lambda i: (0, 0)),
            pl.BlockSpec((1, D1), lambda i: (0, 0)),
            pl.BlockSpec((1, D1), lambda i: (0, 0)),
        ],
        out_specs=pl.BlockSpec((R, D1), lambda i: (i, 0)),
        out_shape=jax.ShapeDtypeStruct((N, D1), jnp.float32),
        compiler_params=pltpu.CompilerParams(dimension_semantics=("arbitrary",)),
    )(t2, stat, cstat, g, b)


def _pool_body(h_ref, batch_ref, out_ref):
    @pl.when(pl.program_id(0) == 0)
    def _():
        out_ref[...] = jnp.full_like(out_ref, -jnp.inf)

    b = batch_ref[...]
    h = h_ref[...]
    gid = lax.broadcasted_iota(jnp.int32, (G, 1), 0)
    acc = out_ref[...]

    def body(g, acc):
        m = jnp.max(jnp.where(b == g, h, -jnp.inf), axis=0, keepdims=True)
        return jnp.where(gid == g, jnp.maximum(acc, m), acc)

    out_ref[...] = lax.fori_loop(0, G, body, acc)


def _pool(h, batch3d):
    return pl.pallas_call(
        _pool_body,
        grid=(NBLK,),
        in_specs=[
            pl.BlockSpec((R, D1), lambda i: (i, 0)),
            pl.BlockSpec((R, 1), lambda i: (i, 0)),
        ],
        out_specs=pl.BlockSpec((G, D1), lambda i: (0, 0)),
        out_shape=jax.ShapeDtypeStruct((G, D1), jnp.float32),
        compiler_params=pltpu.CompilerParams(dimension_semantics=("arbitrary",)),
    )(h, batch3d)


def _head_body(pb_ref, pr_ref, wb_ref, bb_ref, wr_ref, br_ref,
               wbe_ref, bbe_ref, wm_ref, bm_ref, out_ref):
    eb = jnp.maximum(
        jnp.dot(pb_ref[...], wb_ref[...], preferred_element_type=jnp.float32)
        + bb_ref[...], 0.0)
    er = jnp.maximum(
        jnp.dot(pr_ref[...], wr_ref[...], preferred_element_type=jnp.float32)
        + br_ref[...], 0.0)
    cat = jnp.concatenate([eb, er], axis=1)
    hh = jnp.maximum(
        jnp.dot(cat, wbe_ref[...], preferred_element_type=jnp.float32)
        + bbe_ref[...], 0.0)
    z = jnp.dot(hh, wm_ref[...], preferred_element_type=jnp.float32) + bm_ref[...]
    out_ref[...] = 1.0 / (1.0 + jnp.exp(-z))


def _head(pb, pr, wb, bb, wr, br, wbe, bbe, wm, bm):
    return pl.pallas_call(
        _head_body,
        out_shape=jax.ShapeDtypeStruct((G, 1), jnp.float32),
    )(pb, pr, wb, bb, wr, br, wbe, bbe, wm, bm)


def _branch(x, ei, batch, br, p, zeros16, zeros32):
    src = jnp.concatenate([ei[0], jnp.zeros((EPAD - E,), jnp.int32)])
    dst = jnp.concatenate([ei[1], jnp.full((EPAD - E,), N, jnp.int32)])
    h = jnp.pad(x, ((0, 0), (0, 16 - F_IN)))
    w1_1 = jnp.pad(p[br + "_c1_W1"], ((0, 16 - F_IN), (0, 0)))
    batch2d = batch.astype(jnp.int32).reshape(N, 1)

    for i in range(1, 4):
        if i == 1:
            agg = _seg_sum16(h, src, dst, zeros16)
            w1 = w1_1
        else:
            agg = _seg_sum32(h, src, dst, zeros32)
            w1 = p[br + "_c%d_W1" % i]
        t2, stat = _mlp(h, agg, w1,
                        p[br + "_c%d_b1" % i].reshape(1, D1),
                        p[br + "_c%d_W2" % i],
                        p[br + "_c%d_b2" % i].reshape(1, D1))
        cstat = _bnstat(t2, stat)
        h = _bn(t2, stat, cstat,
                p[br + "_bn%d_g" % i].reshape(1, D1),
                p[br + "_bn%d_b" % i].reshape(1, D1))
    return _pool(h, batch2d)


def kernel(data_base, edge_index_base, batch_base,
           data_residual, edge_index_residual, batch_residual, params):
    zeros16 = jnp.zeros((ZR, 16), jnp.float32)
    zeros32 = jnp.zeros((ZR, 32), jnp.float32)
    pb = _branch(data_base, edge_index_base, batch_base, "base", params,
                 zeros16, zeros32)
    pr = _branch(data_residual, edge_index_residual, batch_residual, "res", params,
                 zeros16, zeros32)
    return _head(pb, pr,
                 params["base_Wbr"], params["base_bbr"].reshape(1, D1),
                 params["res_Wbr"], params["res_bbr"].reshape(1, D1),
                 params["W_before"], params["b_before"].reshape(1, D2),
                 params["W_mean"], params["b_mean"].reshape(1, 1))


# speedup vs baseline: 5.9882x; 1.4813x over previous
import functools

import jax
import jax.numpy as jnp
from jax import lax
from jax.experimental import pallas as pl
from jax.experimental.pallas import tpu as pltpu
from jax.experimental.pallas import tpu_sc as plsc

N = 50000
E = 800000
F_IN = 6
D1 = 32
D2 = 16
G = 64

NC = 2
NS = 16
NW = NC * NS
HD = 16
CH = 128
K = 7
S = K * CH
EPAD = 802816
NCHT = EPAD // (NS * CH)
NSUPER = NCHT // K
NACC = 50048
ZR = NACC // NS
R = 2000
NBLK = N // R


def _make_seg_sum():
    mesh = plsc.VectorSubcoreMesh(core_axis_name="c", subcore_axis_name="s",
                                  num_cores=NC, num_subcores=NS)

    @functools.partial(
        pl.kernel,
        out_type=jax.ShapeDtypeStruct((NC, NACC, HD), jnp.float32),
        mesh=mesh,
        scratch_types=[
            pltpu.VMEM((K, 1, CH), jnp.int32),
            pltpu.VMEM((K, 1, CH), jnp.int32),
            pltpu.VMEM((K, 1, CH), jnp.int32),
            pltpu.VMEM((K, 1, CH), jnp.int32),
            pltpu.VMEM((S, HD), jnp.float32),
            pltpu.VMEM((S, HD), jnp.float32),
            pltpu.VMEM_SHARED((NACC, HD), jnp.float32),
            pltpu.SemaphoreType.DMA,
            pltpu.SemaphoreType.DMA,
            pltpu.SemaphoreType.DMA,
            pltpu.SemaphoreType.DMA,
            pltpu.SemaphoreType.DMA,
            pltpu.SemaphoreType.DMA,
        ],
        compiler_params=pltpu.CompilerParams(use_tc_tiling_on_sc=False),
    )
    def seg_sum(h_hbm, src_hbm, dst_hbm, zeros_hbm, out_hbm,
                sidx0, sidx1, didx0, didx1, rows0, rows1, acc,
                si0, si1, sg0, sg1, ss0, ss1):
        c = lax.axis_index("c")
        s = lax.axis_index("s")
        pltpu.sync_copy(zeros_hbm, acc.at[pl.ds(s * ZR, ZR)])
        plsc.subcore_barrier()
        cbase = s * NCHT
        htab = h_hbm.at[c]
        sidx = (sidx0, sidx1)
        didx = (didx0, didx1)
        rows = (rows0, rows1)
        si = (si0, si1)
        sg = (sg0, sg1)
        ss = (ss0, ss1)
        zslab = zeros_hbm.at[pl.ds(0, S)]

        @pl.loop(0, NSUPER, step=2)
        def _(i0):
            for b in range(2):
                i = i0 + b

                @pl.when(i >= 2)
                def _():
                    pltpu.make_async_copy(zslab, rows[b], ss[b]).wait()

                co = cbase + i * K
                d1 = pltpu.async_copy(src_hbm.at[pl.ds(co, K)], sidx[b], si[b])
                d2 = pltpu.async_copy(dst_hbm.at[pl.ds(co, K)], didx[b], si[b])
                d1.wait()
                d2.wait()

                @pl.loop(0, K)
                def _(j):
                    pltpu.async_copy(htab.at[sidx[b].at[j, 0]],
                                     rows[b].at[pl.ds(j * CH, CH)], sg[b])

                pltpu.make_async_copy(zslab, rows[b], sg[b]).wait()

                @pl.loop(0, K)
                def _(j):
                    pltpu.async_copy(rows[b].at[pl.ds(j * CH, CH)],
                                     acc.at[didx[b].at[j, 0]], ss[b], add=True)

        for b in range(2):
            pltpu.make_async_copy(zslab, rows[b], ss[b]).wait()
        plsc.subcore_barrier()
        pltpu.sync_copy(acc.at[pl.ds(s * ZR, ZR)], out_hbm.at[c, pl.ds(s * ZR, ZR)])

    return seg_sum


_make_seg_sum = functools.lru_cache(maxsize=None)(_make_seg_sum)


def _seg_sum(h2, src, dst, z):
    return _make_seg_sum()(h2, src, dst, z)


def _mlp_body(h_ref, agg_ref, w1_ref, b1_ref, w2_ref, b2_ref, t2_ref, stat_ref):
    x = jnp.concatenate([h_ref[0] + agg_ref[0], h_ref[1] + agg_ref[1]], axis=1)
    t = jnp.maximum(
        jnp.dot(x, w1_ref[...], preferred_element_type=jnp.float32) + b1_ref[...], 0.0)
    t2 = jnp.maximum(
        jnp.dot(t, w2_ref[...], preferred_element_type=jnp.float32) + b2_ref[...], 0.0)
    t2_ref[...] = t2
    bsum = jnp.sum(t2, axis=0, keepdims=True)
    row0 = lax.broadcasted_iota(jnp.int32, (8, 1), 0) == 0
    stat_ref[...] = jnp.where(row0, jnp.broadcast_to(bsum, (8, D1)), 0.0)


def _mlp(h, agg, w1, b1, w2, b2):
    din = w1.shape[0]
    return pl.pallas_call(
        _mlp_body,
        grid=(NBLK,),
        in_specs=[
            pl.BlockSpec((NC, R, HD), lambda i: (0, i, 0)),
            pl.BlockSpec((NC, R, HD), lambda i: (0, i, 0)),
            pl.BlockSpec((din, D1), lambda i: (0, 0)),
            pl.BlockSpec((1, D1), lambda i: (0, 0)),
            pl.BlockSpec((D1, D1), lambda i: (0, 0)),
            pl.BlockSpec((1, D1), lambda i: (0, 0)),
        ],
        out_specs=[
            pl.BlockSpec((R, D1), lambda i: (i, 0)),
            pl.BlockSpec((8, D1), lambda i: (i, 0)),
        ],
        out_shape=[
            jax.ShapeDtypeStruct((N, D1), jnp.float32),
            jax.ShapeDtypeStruct((8 * NBLK, D1), jnp.float32),
        ],
        compiler_params=pltpu.CompilerParams(dimension_semantics=("arbitrary",)),
    )(h, agg, w1, b1, w2, b2)


def _bnstat_body(t2_ref, stat_ref, out_ref):
    mean = jnp.sum(stat_ref[...], axis=0, keepdims=True) / N
    d = t2_ref[...] - mean
    bsum = jnp.sum(d * d, axis=0, keepdims=True)
    row0 = lax.broadcasted_iota(jnp.int32, (8, 1), 0) == 0
    out_ref[...] = jnp.where(row0, jnp.broadcast_to(bsum, (8, D1)), 0.0)


def _bnstat(t2, stat):
    return pl.pallas_call(
        _bnstat_body,
        grid=(NBLK,),
        in_specs=[
            pl.BlockSpec((R, D1), lambda i: (i, 0)),
            pl.BlockSpec((8 * NBLK, D1), lambda i: (0, 0)),
        ],
        out_specs=pl.BlockSpec((8, D1), lambda i: (i, 0)),
        out_shape=jax.ShapeDtypeStruct((8 * NBLK, D1), jnp.float32),
        compiler_params=pltpu.CompilerParams(dimension_semantics=("arbitrary",)),
    )(t2, stat)


def _bn_body(t2_ref, stat_ref, cstat_ref, g_ref, b_ref, out_ref):
    mean = jnp.sum(stat_ref[...], axis=0, keepdims=True) / N
    var = jnp.sum(cstat_ref[...], axis=0, keepdims=True) / N
    d = var + 1e-5
    r = lax.rsqrt(d)
    r = r * (1.5 - 0.5 * d * r * r)
    r = r * (1.5 - 0.5 * d * r * r)
    y = (t2_ref[...] - mean) * (r * g_ref[...]) + b_ref[...]
    out_ref[0] = y[:, :HD]
    out_ref[1] = y[:, HD:]


def _bn(t2, stat, cstat, g, b):
    return pl.pallas_call(
        _bn_body,
        grid=(NBLK,),
        in_specs=[
            pl.BlockSpec((R, D1), lambda i: (i, 0)),
            pl.BlockSpec((8 * NBLK, D1), lambda i: (0, 0)),
            pl.BlockSpec((8 * NBLK, D1), lambda i: (0, 0)),
            pl.BlockSpec((1, D1), lambda i: (0, 0)),
            pl.BlockSpec((1, D1), lambda i: (0, 0)),
        ],
        out_specs=pl.BlockSpec((NC, R, HD), lambda i: (0, i, 0)),
        out_shape=jax.ShapeDtypeStruct((NC, N, HD), jnp.float32),
        compiler_params=pltpu.CompilerParams(dimension_semantics=("arbitrary",)),
    )(t2, stat, cstat, g, b)


def _pool_body(h_ref, batch_ref, out_ref):
    @pl.when(pl.program_id(0) == 0)
    def _():
        out_ref[...] = jnp.full_like(out_ref, -jnp.inf)

    b = batch_ref[...]
    h = jnp.concatenate([h_ref[0], h_ref[1]], axis=1)
    gid = lax.broadcasted_iota(jnp.int32, (G, 1), 0)
    acc = out_ref[...]

    def body(g, acc):
        m = jnp.max(jnp.where(b == g, h, -jnp.inf), axis=0, keepdims=True)
        return jnp.where(gid == g, jnp.maximum(acc, m), acc)

    out_ref[...] = lax.fori_loop(0, G, body, acc)


def _pool(h, batch3d):
    return pl.pallas_call(
        _pool_body,
        grid=(NBLK,),
        in_specs=[
            pl.BlockSpec((NC, R, HD), lambda i: (0, i, 0)),
            pl.BlockSpec((R, 1), lambda i: (i, 0)),
        ],
        out_specs=pl.BlockSpec((G, D1), lambda i: (0, 0)),
        out_shape=jax.ShapeDtypeStruct((G, D1), jnp.float32),
        compiler_params=pltpu.CompilerParams(dimension_semantics=("arbitrary",)),
    )(h, batch3d)


def _head_body(pb_ref, pr_ref, wb_ref, bb_ref, wr_ref, br_ref,
               wbe_ref, bbe_ref, wm_ref, bm_ref, out_ref):
    eb = jnp.maximum(
        jnp.dot(pb_ref[...], wb_ref[...], preferred_element_type=jnp.float32)
        + bb_ref[...], 0.0)
    er = jnp.maximum(
        jnp.dot(pr_ref[...], wr_ref[...], preferred_element_type=jnp.float32)
        + br_ref[...], 0.0)
    cat = jnp.concatenate([eb, er], axis=1)
    hh = jnp.maximum(
        jnp.dot(cat, wbe_ref[...], preferred_element_type=jnp.float32)
        + bbe_ref[...], 0.0)
    z = jnp.dot(hh, wm_ref[...], preferred_element_type=jnp.float32) + bm_ref[...]
    out_ref[...] = 1.0 / (1.0 + jnp.exp(-z))


def _head(pb, pr, wb, bb, wr, br, wbe, bbe, wm, bm):
    return pl.pallas_call(
        _head_body,
        out_shape=jax.ShapeDtypeStruct((G, 1), jnp.float32),
    )(pb, pr, wb, bb, wr, br, wbe, bbe, wm, bm)


def _branch(x, ei, batch, br, p, zeros):
    src = jnp.concatenate([ei[0], jnp.zeros((EPAD - E,), jnp.int32)]
                          ).reshape(EPAD // CH, 1, CH)
    dst = jnp.concatenate([ei[1], jnp.full((EPAD - E,), N, jnp.int32)]
                          ).reshape(EPAD // CH, 1, CH)
    h = jnp.stack([jnp.pad(x, ((0, 0), (0, HD - F_IN))),
                   jnp.zeros((N, HD), jnp.float32)])
    w1_1 = jnp.pad(p[br + "_c1_W1"], ((0, D1 - F_IN), (0, 0)))
    batch2d = batch.astype(jnp.int32).reshape(N, 1)

    for i in range(1, 4):
        agg = _seg_sum(h, src, dst, zeros)
        w1 = w1_1 if i == 1 else p[br + "_c%d_W1" % i]
        t2, stat = _mlp(h, agg, w1,
                        p[br + "_c%d_b1" % i].reshape(1, D1),
                        p[br + "_c%d_W2" % i],
                        p[br + "_c%d_b2" % i].reshape(1, D1))
        cstat = _bnstat(t2, stat)
        h = _bn(t2, stat, cstat,
                p[br + "_bn%d_g" % i].reshape(1, D1),
                p[br + "_bn%d_b" % i].reshape(1, D1))
    return _pool(h, batch2d)


def kernel(data_base, edge_index_base, batch_base,
           data_residual, edge_index_residual, batch_residual, params):
    zeros = jnp.zeros((ZR, HD), jnp.float32)
    pb = _branch(data_base, edge_index_base, batch_base, "base", params, zeros)
    pr = _branch(data_residual, edge_index_residual, batch_residual, "res", params,
                 zeros)
    return _head(pb, pr,
                 params["base_Wbr"], params["base_bbr"].reshape(1, D1),
                 params["res_Wbr"], params["res_bbr"].reshape(1, D1),
                 params["W_before"], params["b_before"].reshape(1, D2),
                 params["W_mean"], params["b_mean"].reshape(1, 1))
